# Initial kernel scaffold; baseline (speedup 1.0000x reference)
import jax
import jax.numpy as jnp
from jax.experimental import pallas as pl


def kernel(img_feats, cam2ego, cam_intrinsics, W_dn, b_dn):
    raise NotImplementedError("write your pallas kernel here")


# speedup vs baseline: 1.0288x; 1.0288x over previous
import functools

import jax
import jax.numpy as jnp
from jax.experimental import pallas as pl

_D = 59
_OUT_C = 64
_X = 128
_Y = 128
_Z = 7
_STRIDE = 16.0


def _dense_body(x_ref, w_ref, b_ref, depth_ref, feat_ref):
    xb = x_ref[0]
    w = w_ref[...]
    out = jax.lax.dot_general(xb.astype(jnp.bfloat16), w.astype(jnp.bfloat16),
                              (((0,), (1,)), ((), ())),
                              preferred_element_type=jnp.float32)
    out = out + b_ref[...]
    logits = out[:, :_D]
    m = jnp.max(logits, axis=1, keepdims=True)
    e = jnp.exp(logits - m)
    s = jnp.sum(e, axis=1, keepdims=True)
    depth_ref[0] = e / s
    feat_ref[0] = out[:, _D:_D + _OUT_C]


def _dense_stage(img_feats, W_dn, b_dn):
    B, N, C, H, W = img_feats.shape
    BN = B * N
    HW = H * W
    x = img_feats.reshape(BN, C, HW)
    w_p = jnp.zeros((128, C), jnp.float32).at[:_D + _OUT_C].set(W_dn)
    b_p = jnp.zeros((1, 128), jnp.float32).at[0, :_D + _OUT_C].set(b_dn)
    depth, feat = pl.pallas_call(
        _dense_body,
        grid=(BN,),
        in_specs=[
            pl.BlockSpec((1, C, HW), lambda i: (i, 0, 0)),
            pl.BlockSpec((128, C), lambda i: (0, 0)),
            pl.BlockSpec((1, 128), lambda i: (0, 0)),
        ],
        out_specs=[
            pl.BlockSpec((1, HW, _D), lambda i: (i, 0, 0)),
            pl.BlockSpec((1, HW, _OUT_C), lambda i: (i, 0, 0)),
        ],
        out_shape=[
            jax.ShapeDtypeStruct((BN, HW, _D), jnp.float32),
            jax.ShapeDtypeStruct((BN, HW, _OUT_C), jnp.float32),
        ],
    )(x, w_p, b_p)
    return depth, feat


def kernel(img_feats, cam2ego, cam_intrinsics, W_dn, b_dn):
    B, N, C, H, W = img_feats.shape
    depth, feat = _dense_stage(img_feats, W_dn, b_dn)

    xs3, ys3, zs3 = jnp.meshgrid(jnp.arange(_X), jnp.arange(_Y), jnp.arange(_Z),
                                 indexing='ij')
    coords = jnp.stack([xs3, ys3, zs3], axis=3).astype(jnp.float32)
    lb = jnp.array([-51.2, -51.2, -2.5], dtype=jnp.float32)
    iv = jnp.array([0.8, 0.8, 1.0], dtype=jnp.float32)
    vc = (coords * iv + lb).reshape(-1, 3)
    nv = vc.shape[0]
    vox_homo = jnp.concatenate([vc, jnp.ones((nv, 1), jnp.float32)], axis=1)

    def _proj(e2c_bn, K_bn):
        cam_coords = (e2c_bn @ vox_homo.T).T[:, :3]
        z = cam_coords[:, 2]
        valid_z = z > 0.5
        z_safe = jnp.clip(z, 0.1, None)
        norm2 = cam_coords[:, :2] / z_safe[:, None]
        homo = jnp.concatenate([norm2, jnp.ones((nv, 1), jnp.float32)], axis=1)
        img_coords = (K_bn @ homo.T).T[:, :2]
        fc = img_coords / _STRIDE
        valid = (valid_z & (fc[:, 0] >= 0) & (fc[:, 0] < W)
                 & (fc[:, 1] >= 0) & (fc[:, 1] < H))
        depth_bin = (z - 1.0).astype(jnp.int32)
        valid = valid & (depth_bin >= 0) & (depth_bin < _D)
        u = jnp.clip(fc[:, 0].astype(jnp.int32), 0, W - 1)
        v = jnp.clip(fc[:, 1].astype(jnp.int32), 0, H - 1)
        d = jnp.clip(depth_bin, 0, _D - 1)
        return v * W + u, d, valid

    e2c = jnp.linalg.inv(cam2ego)
    pix, d, valid = jax.vmap(jax.vmap(_proj))(e2c, cam_intrinsics)
    didx = pix * _D + d

    depth4 = depth.reshape(B, N, H * W * _D)
    feat4 = feat.reshape(B, N, H * W, _OUT_C)
    w_s = jnp.take_along_axis(depth4, didx, axis=2) * valid.astype(jnp.float32)
    f_s = jnp.take_along_axis(feat4, pix[..., None], axis=2)
    weighted = f_s * w_s[..., None]
    bev = weighted.reshape(B, N, _X, _Y, _Z, _OUT_C).sum(axis=(1, 4))
    bev = bev.transpose(0, 3, 2, 1)

    depth_out = depth.reshape(B, N, H, W, _D)
    return bev, depth_out


# speedup vs baseline: 14.5624x; 14.1552x over previous
import functools

import jax
import jax.numpy as jnp
from jax import lax
from jax.experimental import pallas as pl
from jax.experimental.pallas import tpu as pltpu
from jax.experimental.pallas import tpu_sc as plsc

_D = 59
_OUT_C = 64
_X = 128
_Y = 128
_Z = 7
_STRIDE = 16.0
_NV = _X * _Y * _Z
_NCOL = _X * _Y
_NW = 32
_CPW = _NCOL // _NW
_FSZ = 704 * _OUT_C
_DSZ = 704 * _D
_DPAD = _DSZ + 8


def _dense_body(x_ref, w_ref, b_ref, depth_ref, feat_ref):
    xb = x_ref[0]
    w = w_ref[...]
    out = jax.lax.dot_general(xb.astype(jnp.bfloat16), w.astype(jnp.bfloat16),
                              (((0,), (1,)), ((), ())),
                              preferred_element_type=jnp.float32)
    out = out + b_ref[...]
    logits = out[:, :_D]
    m = jnp.max(logits, axis=1, keepdims=True)
    e = jnp.exp(logits - m)
    s = jnp.sum(e, axis=1, keepdims=True)
    depth_ref[0] = e / s
    feat_ref[0] = out[:, _D:_D + _OUT_C]


def _dense_stage(img_feats, W_dn, b_dn):
    B, N, C, H, W = img_feats.shape
    BN = B * N
    HW = H * W
    x = img_feats.reshape(BN, C, HW)
    w_p = jnp.zeros((128, C), jnp.float32).at[:_D + _OUT_C].set(W_dn)
    b_p = jnp.zeros((1, 128), jnp.float32).at[0, :_D + _OUT_C].set(b_dn)
    depth, feat = pl.pallas_call(
        _dense_body,
        grid=(BN,),
        in_specs=[
            pl.BlockSpec((1, C, HW), lambda i: (i, 0, 0)),
            pl.BlockSpec((128, C), lambda i: (0, 0)),
            pl.BlockSpec((1, 128), lambda i: (0, 0)),
        ],
        out_specs=[
            pl.BlockSpec((1, HW, _D), lambda i: (i, 0, 0)),
            pl.BlockSpec((1, HW, _OUT_C), lambda i: (i, 0, 0)),
        ],
        out_shape=[
            jax.ShapeDtypeStruct((BN, HW, _D), jnp.float32),
            jax.ShapeDtypeStruct((BN, HW, _OUT_C), jnp.float32),
        ],
    )(x, w_p, b_p)
    return depth, feat


_sc_mesh = plsc.VectorSubcoreMesh(core_axis_name="c", subcore_axis_name="s")


@functools.partial(
    pl.kernel,
    out_type=jax.ShapeDtypeStruct((2, _NW, _OUT_C * _CPW), jnp.float32),
    mesh=_sc_mesh,
    compiler_params=pltpu.CompilerParams(needs_layout_passes=False),
    scratch_types=[
        pltpu.VMEM((_FSZ,), jnp.float32),
        pltpu.VMEM((_DPAD,), jnp.float32),
        pltpu.VMEM((_Z * _CPW,), jnp.int32),
        pltpu.VMEM((_Z * _CPW,), jnp.int32),
        pltpu.VMEM((_OUT_C * _CPW,), jnp.float32),
    ],
)
def _sc_gather(feat_hbm, dpt_hbm, didx_hbm, fb_hbm, out_hbm,
               feat_v, dpt_v, didx_v, fb_v, acc_v):
    wid = lax.axis_index("s") * 2 + lax.axis_index("c")
    zero16 = jnp.zeros((16,), jnp.float32)
    for b in range(2):
        def _zero(i, _):
            acc_v[pl.ds(i * 16, 16)] = zero16
            return None
        lax.fori_loop(0, _OUT_C * _CPW // 16, _zero, None)
        for n in range(6):
            bn = b * 6 + n
            pltpu.sync_copy(feat_hbm.at[bn], feat_v)
            pltpu.sync_copy(dpt_hbm.at[bn], dpt_v)
            pltpu.sync_copy(didx_hbm.at[bn, wid], didx_v)
            pltpu.sync_copy(fb_hbm.at[bn, wid], fb_v)

            def _zbody(z, _):
                def _gbody(g, _):
                    base = z * _CPW + g * 16
                    dvec = didx_v[pl.ds(base, 16)]
                    fbv = fb_v[pl.ds(base, 16)]
                    wvec = plsc.load_gather(dpt_v, [dvec])
                    for ch in range(_OUT_C):
                        fvec = plsc.load_gather(feat_v, [fbv + ch])
                        plsc.addupdate(
                            acc_v.at[pl.ds(ch * _CPW + g * 16, 16)],
                            fvec * wvec)
                    return None
                lax.fori_loop(0, _CPW // 16, _gbody, None)
                return None
            lax.fori_loop(0, _Z, _zbody, None)
        pltpu.sync_copy(acc_v, out_hbm.at[b, wid])


def _projection(cam2ego, cam_intrinsics, H, W):
    xs3, ys3, zs3 = jnp.meshgrid(jnp.arange(_X), jnp.arange(_Y), jnp.arange(_Z),
                                 indexing='ij')
    coords = jnp.stack([xs3, ys3, zs3], axis=3).astype(jnp.float32)
    lb = jnp.array([-51.2, -51.2, -2.5], dtype=jnp.float32)
    iv = jnp.array([0.8, 0.8, 1.0], dtype=jnp.float32)
    vc = (coords * iv + lb).reshape(-1, 3)
    nv = vc.shape[0]
    vox_homo = jnp.concatenate([vc, jnp.ones((nv, 1), jnp.float32)], axis=1)

    def _proj(e2c_bn, K_bn):
        cam_coords = (e2c_bn @ vox_homo.T).T[:, :3]
        z = cam_coords[:, 2]
        valid_z = z > 0.5
        z_safe = jnp.clip(z, 0.1, None)
        norm2 = cam_coords[:, :2] / z_safe[:, None]
        homo = jnp.concatenate([norm2, jnp.ones((nv, 1), jnp.float32)], axis=1)
        img_coords = (K_bn @ homo.T).T[:, :2]
        fc = img_coords / _STRIDE
        valid = (valid_z & (fc[:, 0] >= 0) & (fc[:, 0] < W)
                 & (fc[:, 1] >= 0) & (fc[:, 1] < H))
        depth_bin = (z - 1.0).astype(jnp.int32)
        valid = valid & (depth_bin >= 0) & (depth_bin < _D)
        u = jnp.clip(fc[:, 0].astype(jnp.int32), 0, W - 1)
        v = jnp.clip(fc[:, 1].astype(jnp.int32), 0, H - 1)
        d = jnp.clip(depth_bin, 0, _D - 1)
        return v * W + u, d, valid

    e2c = jnp.linalg.inv(cam2ego)
    pix, d, valid = jax.vmap(jax.vmap(_proj))(e2c, cam_intrinsics)
    return pix, d, valid


def _reorder(a):
    BN = a.shape[0]
    return (a.reshape(BN, _X, _Y, _Z)
             .transpose(0, 3, 2, 1)
             .reshape(BN, _Z, _NW, _CPW)
             .transpose(0, 2, 1, 3)
             .reshape(BN, _NW, _Z * _CPW))


def kernel(img_feats, cam2ego, cam_intrinsics, W_dn, b_dn):
    B, N, C, H, W = img_feats.shape
    BN = B * N
    depth, feat = _dense_stage(img_feats, W_dn, b_dn)

    pix, d, valid = _projection(cam2ego, cam_intrinsics, H, W)
    pix = pix.reshape(BN, _NV)
    d = d.reshape(BN, _NV)
    valid = valid.reshape(BN, _NV)
    didx = jnp.where(valid, pix * _D + d, _DSZ).astype(jnp.int32)
    fb = (pix * _OUT_C).astype(jnp.int32)

    didx_r = _reorder(didx)
    fb_r = _reorder(fb)

    feat_flat = feat.reshape(BN, _FSZ)
    dpt_flat = jnp.pad(depth.reshape(BN, _DSZ), ((0, 0), (0, _DPAD - _DSZ)))

    out = _sc_gather(feat_flat, dpt_flat, didx_r, fb_r)
    bev = (out.reshape(B, _NW, _OUT_C, _CPW)
              .transpose(0, 2, 1, 3)
              .reshape(B, _OUT_C, _Y, _X))

    depth_out = depth.reshape(B, N, H, W, _D)
    return bev, depth_out


# speedup vs baseline: 33.3160x; 2.2878x over previous
import functools

import jax
import jax.numpy as jnp
from jax import lax
from jax.experimental import pallas as pl
from jax.experimental.pallas import tpu as pltpu
from jax.experimental.pallas import tpu_sc as plsc

_D = 59
_OUT_C = 64
_X = 128
_Y = 128
_Z = 7
_STRIDE = 16.0
_NV = _X * _Y * _Z
_NCOL = _X * _Y
_NW = 32
_CPW = _NCOL // _NW
_FSTR = _OUT_C + 1
_FSZ = 704 * _FSTR
_DSZ = 704 * _D
_DPAD = _DSZ + 8


def _dense_body(x_ref, w_ref, b_ref, depth_ref, feat_ref):
    xb = x_ref[0]
    w = w_ref[...]
    out = jax.lax.dot_general(xb.astype(jnp.bfloat16), w.astype(jnp.bfloat16),
                              (((0,), (1,)), ((), ())),
                              preferred_element_type=jnp.float32)
    out = out + b_ref[...]
    logits = out[:, :_D]
    m = jnp.max(logits, axis=1, keepdims=True)
    e = jnp.exp(logits - m)
    s = jnp.sum(e, axis=1, keepdims=True)
    depth_ref[0] = e / s
    feat_ref[0] = out[:, _D:_D + _OUT_C]


def _dense_stage(img_feats, W_dn, b_dn):
    B, N, C, H, W = img_feats.shape
    BN = B * N
    HW = H * W
    x = img_feats.reshape(BN, C, HW)
    w_p = jnp.zeros((128, C), jnp.float32).at[:_D + _OUT_C].set(W_dn)
    b_p = jnp.zeros((1, 128), jnp.float32).at[0, :_D + _OUT_C].set(b_dn)
    depth, feat = pl.pallas_call(
        _dense_body,
        grid=(BN,),
        in_specs=[
            pl.BlockSpec((1, C, HW), lambda i: (i, 0, 0)),
            pl.BlockSpec((128, C), lambda i: (0, 0)),
            pl.BlockSpec((1, 128), lambda i: (0, 0)),
        ],
        out_specs=[
            pl.BlockSpec((1, HW, _D), lambda i: (i, 0, 0)),
            pl.BlockSpec((1, HW, _OUT_C), lambda i: (i, 0, 0)),
        ],
        out_shape=[
            jax.ShapeDtypeStruct((BN, HW, _D), jnp.float32),
            jax.ShapeDtypeStruct((BN, HW, _OUT_C), jnp.float32),
        ],
    )(x, w_p, b_p)
    return depth, feat


_sc_mesh = plsc.VectorSubcoreMesh(core_axis_name="c", subcore_axis_name="s")


@functools.partial(
    pl.kernel,
    out_type=jax.ShapeDtypeStruct((2, _NW, _OUT_C * _CPW), jnp.float32),
    mesh=_sc_mesh,
    compiler_params=pltpu.CompilerParams(needs_layout_passes=False),
    scratch_types=[
        pltpu.VMEM((_FSZ,), jnp.float32),
        pltpu.VMEM((_DPAD,), jnp.float32),
        pltpu.VMEM((_Z * _CPW,), jnp.int32),
        pltpu.VMEM((_Z * _CPW,), jnp.int32),
        pltpu.VMEM((_OUT_C * _CPW,), jnp.float32),
    ],
)
def _sc_gather(feat_hbm, dpt_hbm, didx_hbm, fb_hbm, out_hbm,
               feat_v, dpt_v, didx_v, fb_v, acc_v):
    wid = lax.axis_index("s") * 2 + lax.axis_index("c")
    zero16 = jnp.zeros((16,), jnp.float32)
    for b in range(2):
        def _zero(i, _):
            acc_v[pl.ds(i * 16, 16)] = zero16
            return None
        lax.fori_loop(0, _OUT_C * _CPW // 16, _zero, None)
        def _nbody(n, _):
            bn = b * 6 + n
            pltpu.sync_copy(feat_hbm.at[bn], feat_v)
            pltpu.sync_copy(dpt_hbm.at[bn], dpt_v)
            pltpu.sync_copy(didx_hbm.at[bn, wid], didx_v)
            pltpu.sync_copy(fb_hbm.at[bn, wid], fb_v)

            def _zbody(z, _):
                @plsc.parallel_loop(0, _CPW // 16, 1)
                def _gbody(g):
                    base = z * _CPW + g * 16
                    dvec = didx_v[pl.ds(base, 16)]
                    fbv = fb_v[pl.ds(base, 16)]
                    wvec = plsc.load_gather(dpt_v, [dvec])
                    for ch in range(_OUT_C):
                        fvec = plsc.load_gather(feat_v, [fbv + ch])
                        plsc.addupdate(
                            acc_v.at[pl.ds(ch * _CPW + g * 16, 16)],
                            fvec * wvec)
                return None
            lax.fori_loop(0, _Z, _zbody, None)
            return None
        lax.fori_loop(0, 6, _nbody, None)
        pltpu.sync_copy(acc_v, out_hbm.at[b, wid])


def _make_perm():
    import numpy as np
    j = np.arange(_NV)
    t = j // (_Z * _CPW)
    z = (j // _CPW) % _Z
    lcol = j % _CPW
    col = t * _CPW + lcol
    y = col // _X
    x = col % _X
    return x * (_Y * _Z) + y * _Z + z


_PERM = _make_perm()


def _projection(cam2ego, cam_intrinsics, H, W):
    xs3, ys3, zs3 = jnp.meshgrid(jnp.arange(_X), jnp.arange(_Y), jnp.arange(_Z),
                                 indexing='ij')
    coords = jnp.stack([xs3, ys3, zs3], axis=3).astype(jnp.float32)
    lb = jnp.array([-51.2, -51.2, -2.5], dtype=jnp.float32)
    iv = jnp.array([0.8, 0.8, 1.0], dtype=jnp.float32)
    vc = (coords * iv + lb).reshape(-1, 3)[_PERM]
    nv = vc.shape[0]
    vox_homo = jnp.concatenate([vc, jnp.ones((nv, 1), jnp.float32)], axis=1)

    def _proj(e2c_bn, K_bn):
        cam_coords = (e2c_bn @ vox_homo.T).T[:, :3]
        z = cam_coords[:, 2]
        valid_z = z > 0.5
        z_safe = jnp.clip(z, 0.1, None)
        norm2 = cam_coords[:, :2] / z_safe[:, None]
        homo = jnp.concatenate([norm2, jnp.ones((nv, 1), jnp.float32)], axis=1)
        img_coords = (K_bn @ homo.T).T[:, :2]
        fc = img_coords / _STRIDE
        valid = (valid_z & (fc[:, 0] >= 0) & (fc[:, 0] < W)
                 & (fc[:, 1] >= 0) & (fc[:, 1] < H))
        depth_bin = (z - 1.0).astype(jnp.int32)
        valid = valid & (depth_bin >= 0) & (depth_bin < _D)
        u = jnp.clip(fc[:, 0].astype(jnp.int32), 0, W - 1)
        v = jnp.clip(fc[:, 1].astype(jnp.int32), 0, H - 1)
        d = jnp.clip(depth_bin, 0, _D - 1)
        return v * W + u, d, valid

    e2c = jnp.linalg.inv(cam2ego)
    pix, d, valid = jax.vmap(jax.vmap(_proj))(e2c, cam_intrinsics)
    return pix, d, valid


def kernel(img_feats, cam2ego, cam_intrinsics, W_dn, b_dn):
    B, N, C, H, W = img_feats.shape
    BN = B * N
    depth, feat = _dense_stage(img_feats, W_dn, b_dn)

    pix, d, valid = _projection(cam2ego, cam_intrinsics, H, W)
    pix = pix.reshape(BN, _NV)
    d = d.reshape(BN, _NV)
    valid = valid.reshape(BN, _NV)
    didx_r = jnp.where(valid, pix * _D + d, _DSZ).astype(jnp.int32) \
                .reshape(BN, _NW, _Z * _CPW)
    fb_r = (pix * _FSTR).astype(jnp.int32).reshape(BN, _NW, _Z * _CPW)

    feat_flat = jnp.pad(feat, ((0, 0), (0, 0), (0, _FSTR - _OUT_C))) \
                   .reshape(BN, _FSZ)
    dpt_flat = jnp.pad(depth.reshape(BN, _DSZ), ((0, 0), (0, _DPAD - _DSZ)))

    out = _sc_gather(feat_flat, dpt_flat, didx_r, fb_r)
    bev = (out.reshape(B, _NW, _OUT_C, _CPW)
              .transpose(0, 2, 1, 3)
              .reshape(B, _OUT_C, _Y, _X))

    depth_out = depth.reshape(B, N, H, W, _D)
    return bev, depth_out
